# baseline (speedup 1.0000x reference)
import math

import jax
import jax.numpy as jnp
from jax.experimental import pallas as pl

_A = 16
_ACT = 8
_DIM = 128
_G = 25
_ROWS = _G * _A


def _fused_kernel(h_ref, pol_ref, act_ref, obs_ref, wk_ref, bk_ref, wq_ref,
                  bq_ref, noise_ref, out_ref, w_ref):
    h = h_ref[:]
    k = jax.lax.dot_general(h, wk_ref[:], (((1,), (1,)), ((), ())),
                            preferred_element_type=jnp.float32) + bk_ref[:]
    q = jax.lax.dot_general(h, wq_ref[:], (((1,), (1,)), ((), ())),
                            preferred_element_type=jnp.float32) + bq_ref[:]
    qg = q.reshape(_G, _A, _DIM)
    kg = k.reshape(_G, _A, _DIM)
    score = jax.lax.dot_general(qg, kg, (((2,), (2,)), ((0,), (0,))),
                                preferred_element_type=jnp.float32)
    score = score * (1.0 / math.sqrt(_DIM))
    m = jnp.max(score, axis=-1, keepdims=True)
    e = jnp.exp(score - m)
    w = e / jnp.sum(e, axis=-1, keepdims=True)

    pol = pol_ref[:].reshape(_G, _A, _ACT)
    act = act_ref[:].reshape(_G, _A, _ACT)
    z = (w[..., None] * (act[:, None, :, :] - pol[:, None, :, :])
         + pol[:, None, :, :] + noise_ref[:])
    s = jnp.sum(z, axis=2)
    zf = (pol[:, None, :, :] + s[:, :, None, :] - z) * (1.0 / _A)

    obs = obs_ref[:].reshape(_G, _A, _DIM)
    out_ref[:, :, :_DIM] = jnp.broadcast_to(
        obs[:, None, :, :], (_G, _A, _A, _DIM)).reshape(_ROWS, _A, _DIM)
    out_ref[:, :, _DIM:] = zf.reshape(_ROWS, _A, _ACT)
    w_ref[:] = w.reshape(_ROWS, _A, 1)


def kernel(h, policies, actions, obs_proc, Wk, bk, Wq, bq, edge_index):
    n = h.shape[0]
    b = n // _A
    steps = b // _G
    noise = jax.random.normal(jax.random.key(42), (b, _A, _A, _ACT),
                              dtype=jnp.float32) * 0.1
    out, w_out = pl.pallas_call(
        _fused_kernel,
        grid=(steps,),
        in_specs=[
            pl.BlockSpec((_ROWS, _DIM), lambda i: (i, 0)),
            pl.BlockSpec((_ROWS, _ACT), lambda i: (i, 0)),
            pl.BlockSpec((_ROWS, _ACT), lambda i: (i, 0)),
            pl.BlockSpec((_ROWS, _DIM), lambda i: (i, 0)),
            pl.BlockSpec((_DIM, _DIM), lambda i: (0, 0)),
            pl.BlockSpec((1, _DIM), lambda i: (0, 0)),
            pl.BlockSpec((_DIM, _DIM), lambda i: (0, 0)),
            pl.BlockSpec((1, _DIM), lambda i: (0, 0)),
            pl.BlockSpec((_G, _A, _A, _ACT), lambda i: (i, 0, 0, 0)),
        ],
        out_specs=[
            pl.BlockSpec((_ROWS, _A, _DIM + _ACT), lambda i: (i, 0, 0)),
            pl.BlockSpec((_ROWS, _A, 1), lambda i: (i, 0, 0)),
        ],
        out_shape=[
            jax.ShapeDtypeStruct((n, _A, _DIM + _ACT), jnp.float32),
            jax.ShapeDtypeStruct((n, _A, 1), jnp.float32),
        ],
    )(h, policies, actions, obs_proc, Wk, bk.reshape(1, _DIM), Wq,
      bq.reshape(1, _DIM), noise)
    return out, w_out


# speedup vs baseline: 3.9044x; 3.9044x over previous
import math

import jax
import jax.numpy as jnp
from jax.experimental import pallas as pl

_A = 16
_ACT = 8
_DIM = 128
_G = 25
_ROWS = _G * _A


def _fused_kernel(h_ref, pol_ref, act_ref, obs_ref, wk_ref, bk_ref, wq_ref,
                  bq_ref, noise_ref, out_ref, w_ref):
    f32 = jnp.float32
    h = h_ref[:]
    k = jax.lax.dot_general(h, wk_ref[:], (((1,), (1,)), ((), ())),
                            preferred_element_type=f32) + bk_ref[:]
    q = jax.lax.dot_general(h, wq_ref[:], (((1,), (1,)), ((), ())),
                            preferred_element_type=f32) + bq_ref[:]
    qg = q.reshape(_G, _A, _DIM)
    kg = k.reshape(_G, _A, _DIM)
    score = jax.lax.dot_general(qg, kg, (((2,), (2,)), ((0,), (0,))),
                                preferred_element_type=f32)
    score = score * (1.0 / math.sqrt(_DIM))
    m = jnp.max(score, axis=-1, keepdims=True)
    e = jnp.exp(score - m)
    w = e / jnp.sum(e, axis=-1, keepdims=True)

    lane = jax.lax.broadcasted_iota(jnp.int32, (_A, _DIM), 1)
    jidx = jax.lax.broadcasted_iota(jnp.int32, (_A, _DIM), 0)
    rep = (lane // _ACT == jidx).astype(f32)
    l1 = jax.lax.broadcasted_iota(jnp.int32, (_DIM, _DIM), 0)
    l2 = jax.lax.broadcasted_iota(jnp.int32, (_DIM, _DIM), 1)
    seg = (l1 % _ACT == l2 % _ACT).astype(f32)

    w_exp = jax.lax.dot_general(w, rep, (((2,), (0,)), ((), ())),
                                preferred_element_type=f32)
    pol = pol_ref[:].reshape(_G, _DIM)
    act = act_ref[:].reshape(_G, _DIM)
    z = (w_exp * (act - pol)[:, None, :] + pol[:, None, :]
         + noise_ref[:])
    s_exp = jax.lax.dot_general(z, seg, (((2,), (0,)), ((), ())),
                                preferred_element_type=f32)
    zf = (pol[:, None, :] + s_exp - z) * (1.0 / _A)

    obs = obs_ref[:].reshape(_G, _A, _DIM)
    rolled = [obs[:, 0]] + [jnp.roll(obs[:, j], 8 * j, axis=-1)
                            for j in range(1, _A)]
    zero = jnp.zeros((_G, _DIM), f32)
    lane17 = jax.lax.broadcasted_iota(jnp.int32, (_A + 1, _DIM), 1)
    t17 = jax.lax.broadcasted_iota(jnp.int32, (_A + 1, _DIM), 0)
    left_m = (lane17 // _ACT + 1 < t17).astype(f32)
    right_m = (lane17 // _ACT >= t17).astype(f32)
    zf_m = (lane17 // _ACT + 1 == t17).astype(f32)
    prev = jnp.stack([zero] + rolled, axis=1)
    cur = jnp.stack(rolled + [zero], axis=1)
    obs_pat = prev * left_m + cur * right_m
    out = (obs_pat[:, None, :, :]
           + zf[:, :, None, :] * zf_m)
    out_ref[:] = out.reshape(_ROWS, _A + 1, _DIM)
    w_ref[:] = w


def kernel(h, policies, actions, obs_proc, Wk, bk, Wq, bq, edge_index):
    n = h.shape[0]
    b = n // _A
    steps = b // _G
    noise = jax.random.normal(jax.random.key(42), (b, _A, _A, _ACT),
                              dtype=jnp.float32) * 0.1
    out, w_out = pl.pallas_call(
        _fused_kernel,
        grid=(steps,),
        in_specs=[
            pl.BlockSpec((_ROWS, _DIM), lambda i: (i, 0)),
            pl.BlockSpec((1, _G, _DIM), lambda i: (i, 0, 0)),
            pl.BlockSpec((1, _G, _DIM), lambda i: (i, 0, 0)),
            pl.BlockSpec((_ROWS, _DIM), lambda i: (i, 0)),
            pl.BlockSpec((_DIM, _DIM), lambda i: (0, 0)),
            pl.BlockSpec((1, _DIM), lambda i: (0, 0)),
            pl.BlockSpec((_DIM, _DIM), lambda i: (0, 0)),
            pl.BlockSpec((1, _DIM), lambda i: (0, 0)),
            pl.BlockSpec((_G, _A, _DIM), lambda i: (i, 0, 0)),
        ],
        out_specs=[
            pl.BlockSpec((_ROWS, _A + 1, _DIM), lambda i: (i, 0, 0)),
            pl.BlockSpec((_G, _A, _A), lambda i: (i, 0, 0)),
        ],
        out_shape=[
            jax.ShapeDtypeStruct((n, _A + 1, _DIM), jnp.float32),
            jax.ShapeDtypeStruct((b, _A, _A), jnp.float32),
        ],
    )(h, policies.reshape(steps, _G, _DIM), actions.reshape(steps, _G, _DIM),
      obs_proc,
      Wk, bk.reshape(1, _DIM), Wq, bq.reshape(1, _DIM),
      noise.reshape(b, _A, _DIM))
    obs_final = out.reshape(n, _A, _DIM + _ACT)
    return obs_final, w_out.reshape(n, _A, 1)
